# baseline (speedup 1.0000x reference)
import functools

import jax
import jax.numpy as jnp
from jax import lax
from jax.experimental import pallas as pl
from jax.experimental.pallas import tpu as pltpu
from jax.experimental.pallas import tpu_sc as plsc

D_MODEL = 128
CHUNK = 128
NBUF = 5


@functools.lru_cache(maxsize=None)
def _build_gather(hist: int, batch: int, n_emb: int):
    info = plsc.get_sparse_core_info()
    nc, ns = info.num_cores, info.num_subcores
    nw = nc * ns
    b_per_w = batch // nw
    cpw = b_per_w // CHUNK
    n_chunks = hist * cpw
    assert batch % (nw * CHUNK) == 0 and n_chunks % NBUF == 0
    n_groups = n_chunks // NBUF

    mesh = plsc.VectorSubcoreMesh(core_axis_name="c", subcore_axis_name="s")

    @functools.partial(
        pl.kernel,
        mesh=mesh,
        out_type=jax.ShapeDtypeStruct((hist * batch, D_MODEL), jnp.float32),
        scratch_types=(
            [pltpu.VMEM((hist, b_per_w), jnp.int32)]
            + [pltpu.VMEM_SHARED((n_emb, D_MODEL), jnp.float32)]
            + [pltpu.VMEM((CHUNK, D_MODEL), jnp.float32) for _ in range(NBUF)]
            + [pltpu.SemaphoreType.DMA for _ in range(2 * NBUF)]
        ),
        compiler_params=pltpu.CompilerParams(use_tc_tiling_on_sc=True),
    )
    def gather(t_hbm, table_hbm, out_hbm, idx_v, table_sh, *bufs_and_sems):
        rows = bufs_and_sems[:NBUF]
        gsem = bufs_and_sems[NBUF:2 * NBUF]
        osem = bufs_and_sems[2 * NBUF:]

        wid = lax.axis_index("s") * nc + lax.axis_index("c")
        col0 = wid * b_per_w

        @pl.when(lax.axis_index("s") == 0)
        def _():
            pltpu.sync_copy(table_hbm, table_sh)

        pltpu.sync_copy(t_hbm.at[:, pl.ds(col0, b_per_w)], idx_v)
        plsc.subcore_barrier()

        def out_row0(c):
            h = c // cpw
            q = c - h * cpw
            return h * batch + col0 + q * CHUNK

        def start_gather(c, b):
            h = c // cpw
            q = c - h * cpw
            pltpu.async_copy(
                table_sh.at[idx_v.at[h, pl.ds(q * CHUNK, CHUNK)]],
                rows[b], gsem[b])

        def wait_gather(b):
            pltpu.make_async_copy(
                table_sh.at[pl.ds(0, CHUNK)], rows[b], gsem[b]).wait()

        def start_out(c, b):
            pltpu.async_copy(
                rows[b], out_hbm.at[pl.ds(out_row0(c), CHUNK)], osem[b])

        def wait_out(c, b):
            pltpu.make_async_copy(
                rows[b], out_hbm.at[pl.ds(out_row0(c), CHUNK)], osem[b]).wait()

        for b in range(NBUF):
            start_gather(b, b)

        def group_body(g, carry):
            c0 = g * NBUF
            for b in range(NBUF):
                wait_gather(b)
                start_out(c0 + b, b)
            for b in range(NBUF):
                wait_out(c0 + b, b)
                start_gather(c0 + NBUF + b, b)
            return carry

        lax.fori_loop(0, n_groups - 1, group_body, 0)

        c0 = (n_groups - 1) * NBUF
        for b in range(NBUF):
            wait_gather(b)
            start_out(c0 + b, b)
        for b in range(NBUF):
            wait_out(c0 + b, b)

    return gather


def kernel(t, pos_embedding):
    b, h = t.shape
    t_hmaj = jnp.swapaxes(t, 0, 1).astype(jnp.int32)
    out = _build_gather(h, b, pos_embedding.shape[0])(t_hmaj, pos_embedding)
    return jnp.swapaxes(out.reshape(h, b, D_MODEL), 0, 1)


# speedup vs baseline: 1.0172x; 1.0172x over previous
import functools

import jax
import jax.numpy as jnp
from jax import lax
from jax.experimental import pallas as pl
from jax.experimental.pallas import tpu as pltpu
from jax.experimental.pallas import tpu_sc as plsc

D_MODEL = 128
CHUNK = 64
NBUF = 10


@functools.lru_cache(maxsize=None)
def _build_gather(hist: int, batch: int, n_emb: int):
    info = plsc.get_sparse_core_info()
    nc, ns = info.num_cores, info.num_subcores
    nw = nc * ns
    b_per_w = batch // nw
    cpw = b_per_w // CHUNK
    n_chunks = hist * cpw
    assert batch % (nw * CHUNK) == 0 and n_chunks % NBUF == 0
    n_groups = n_chunks // NBUF

    mesh = plsc.VectorSubcoreMesh(core_axis_name="c", subcore_axis_name="s")

    @functools.partial(
        pl.kernel,
        mesh=mesh,
        out_type=jax.ShapeDtypeStruct((hist * batch, D_MODEL), jnp.float32),
        scratch_types=(
            [pltpu.VMEM((hist, b_per_w), jnp.int32)]
            + [pltpu.VMEM_SHARED((n_emb, D_MODEL), jnp.float32)]
            + [pltpu.VMEM((CHUNK, D_MODEL), jnp.float32) for _ in range(NBUF)]
            + [pltpu.SemaphoreType.DMA for _ in range(2 * NBUF)]
        ),
        compiler_params=pltpu.CompilerParams(use_tc_tiling_on_sc=True),
    )
    def gather(t_hbm, table_hbm, out_hbm, idx_v, table_sh, *bufs_and_sems):
        rows = bufs_and_sems[:NBUF]
        gsem = bufs_and_sems[NBUF:2 * NBUF]
        osem = bufs_and_sems[2 * NBUF:]

        wid = lax.axis_index("s") * nc + lax.axis_index("c")
        col0 = wid * b_per_w

        @pl.when(lax.axis_index("s") == 0)
        def _():
            pltpu.sync_copy(table_hbm, table_sh)

        pltpu.sync_copy(t_hbm.at[:, pl.ds(col0, b_per_w)], idx_v)
        plsc.subcore_barrier()

        def out_row0(c):
            h = c // cpw
            q = c - h * cpw
            return h * batch + col0 + q * CHUNK

        def start_gather(c, b):
            h = c // cpw
            q = c - h * cpw
            pltpu.async_copy(
                table_sh.at[idx_v.at[h, pl.ds(q * CHUNK, CHUNK)]],
                rows[b], gsem[b])

        def wait_gather(b):
            pltpu.make_async_copy(
                table_sh.at[pl.ds(0, CHUNK)], rows[b], gsem[b]).wait()

        def start_out(c, b):
            pltpu.async_copy(
                rows[b], out_hbm.at[pl.ds(out_row0(c), CHUNK)], osem[b])

        def wait_out(c, b):
            pltpu.make_async_copy(
                rows[b], out_hbm.at[pl.ds(out_row0(c), CHUNK)], osem[b]).wait()

        for b in range(NBUF):
            start_gather(b, b)

        def group_body(g, carry):
            c0 = g * NBUF
            for b in range(NBUF):
                wait_gather(b)
                start_out(c0 + b, b)
            for b in range(NBUF):
                wait_out(c0 + b, b)
                start_gather(c0 + NBUF + b, b)
            return carry

        lax.fori_loop(0, n_groups - 1, group_body, 0)

        c0 = (n_groups - 1) * NBUF
        for b in range(NBUF):
            wait_gather(b)
            start_out(c0 + b, b)
        for b in range(NBUF):
            wait_out(c0 + b, b)

    return gather


def kernel(t, pos_embedding):
    b, h = t.shape
    t_hmaj = jnp.swapaxes(t, 0, 1).astype(jnp.int32)
    out = _build_gather(h, b, pos_embedding.shape[0])(t_hmaj, pos_embedding)
    return jnp.swapaxes(out.reshape(h, b, D_MODEL), 0, 1)


# speedup vs baseline: 1.0220x; 1.0047x over previous
import functools

import jax
import jax.numpy as jnp
from jax import lax
from jax.experimental import pallas as pl
from jax.experimental.pallas import tpu as pltpu
from jax.experimental.pallas import tpu_sc as plsc

D_MODEL = 128
CHUNK = 64
NBUF = 10


@functools.lru_cache(maxsize=None)
def _build_gather(hist: int, batch: int, n_emb: int):
    info = plsc.get_sparse_core_info()
    nc, ns = info.num_cores, info.num_subcores
    nw = nc * ns
    b_per_w = batch // nw
    cpw = b_per_w // CHUNK
    n_chunks = hist * cpw
    assert batch % (nw * CHUNK) == 0 and n_chunks % NBUF == 0
    n_groups = n_chunks // NBUF

    mesh = plsc.VectorSubcoreMesh(core_axis_name="c", subcore_axis_name="s")

    @functools.partial(
        pl.kernel,
        mesh=mesh,
        out_type=jax.ShapeDtypeStruct((hist * batch, D_MODEL), jnp.float32),
        scratch_types=(
            [pltpu.VMEM((hist, b_per_w), jnp.int32)]
            + [pltpu.VMEM_SHARED((n_emb, D_MODEL), jnp.float32)]
            + [pltpu.VMEM((CHUNK, D_MODEL), jnp.float32) for _ in range(NBUF)]
            + [pltpu.SemaphoreType.DMA for _ in range(2 * NBUF + 1)]
        ),
        compiler_params=pltpu.CompilerParams(use_tc_tiling_on_sc=True),
    )
    def gather(t_hbm, table_hbm, out_hbm, idx_v, table_sh, *bufs_and_sems):
        rows = bufs_and_sems[:NBUF]
        gsem = bufs_and_sems[NBUF:2 * NBUF]
        osem = bufs_and_sems[2 * NBUF:3 * NBUF]
        isem = bufs_and_sems[3 * NBUF]

        sid = lax.axis_index("s")
        wid = sid * nc + lax.axis_index("c")
        col0 = wid * b_per_w

        pltpu.async_copy(t_hbm.at[:, pl.ds(col0, b_per_w)], idx_v, isem)

        tail0 = (n_emb // 128) * 128 if n_emb % 128 else n_emb - 128
        @pl.when(sid < tail0 // 128)
        def _():
            pltpu.sync_copy(table_hbm.at[pl.ds(sid * 128, 128)],
                            table_sh.at[pl.ds(sid * 128, 128)])

        @pl.when(sid == tail0 // 128)
        def _():
            pltpu.sync_copy(table_hbm.at[pl.ds(tail0, n_emb - tail0)],
                            table_sh.at[pl.ds(tail0, n_emb - tail0)])

        pltpu.make_async_copy(t_hbm.at[:, pl.ds(col0, b_per_w)], idx_v, isem).wait()
        plsc.subcore_barrier()

        def out_row0(c):
            h = c // cpw
            q = c - h * cpw
            return h * batch + col0 + q * CHUNK

        def start_gather(c, b):
            h = c // cpw
            q = c - h * cpw
            pltpu.async_copy(
                table_sh.at[idx_v.at[h, pl.ds(q * CHUNK, CHUNK)]],
                rows[b], gsem[b])

        def wait_gather(b):
            pltpu.make_async_copy(
                table_sh.at[pl.ds(0, CHUNK)], rows[b], gsem[b]).wait()

        def start_out(c, b):
            pltpu.async_copy(
                rows[b], out_hbm.at[pl.ds(out_row0(c), CHUNK)], osem[b])

        def wait_out(c, b):
            pltpu.make_async_copy(
                rows[b], out_hbm.at[pl.ds(out_row0(c), CHUNK)], osem[b]).wait()

        for b in range(NBUF):
            start_gather(b, b)

        def group_body(g, carry):
            c0 = g * NBUF
            for b in range(NBUF):
                wait_gather(b)
                start_out(c0 + b, b)
            for b in range(NBUF):
                wait_out(c0 + b, b)
                start_gather(c0 + NBUF + b, b)
            return carry

        lax.fori_loop(0, n_groups - 1, group_body, 0)

        c0 = (n_groups - 1) * NBUF
        for b in range(NBUF):
            wait_gather(b)
            start_out(c0 + b, b)
        for b in range(NBUF):
            wait_out(c0 + b, b)

    return gather


def kernel(t, pos_embedding):
    b, h = t.shape
    t_hmaj = jnp.swapaxes(t, 0, 1).astype(jnp.int32)
    out = _build_gather(h, b, pos_embedding.shape[0])(t_hmaj, pos_embedding)
    return jnp.swapaxes(out.reshape(h, b, D_MODEL), 0, 1)
